# baseline (speedup 1.0000x reference)
import functools

import jax
import jax.numpy as jnp
from jax import lax
from jax.experimental import pallas as pl
from jax.experimental.pallas import tpu as pltpu
from jax.experimental.pallas import tpu_sc as plsc

T = 1024
B = 32
NSLOT = 3
F = 512
DV = 128
R = B * NSLOT
NCORE, NSUB = 2, 16
NW = NCORE * NSUB
T_PER_W = T // NW
TB = 16


def _prep_body(x_ref, vel_ref, ctrl_ref, cidx_ref, table2_ref, pe2_ref):
    slot = lax.broadcasted_iota(jnp.int32, (1, R), 1) % NSLOT
    cidx_ref[...] = x_ref[...] + jnp.where(slot == 0, 0, DV)

    vel = vel_ref[...]
    ctrl = ctrl_ref[...]
    vn = jnp.max(jnp.abs(vel), axis=1, keepdims=True)
    cn = jnp.max(jnp.abs(ctrl), axis=1, keepdims=True)
    vscale = jnp.where(vn > 1.0, 1.0 / vn, 1.0)
    cscale = jnp.where(cn > 127.0, 127.0 / cn, 1.0)
    table2_ref[0:DV, :] = vel * (2.0 * vscale)
    table2_ref[DV:2 * DV, :] = ctrl * (2.0 * cscale)

    pos = lax.broadcasted_iota(jnp.int32, (T, F), 0).astype(jnp.float32)
    col = lax.broadcasted_iota(jnp.int32, (T, F), 1)
    colmod = col % 2
    ang = pos * jnp.exp((col - colmod).astype(jnp.float32) * (-4.0 / F))
    pe2_ref[...] = 2.0 * jnp.where(colmod == 0, jnp.sin(ang), jnp.cos(ang))


def _prep(x2, vel_table, ctrl_table):
    return pl.pallas_call(
        _prep_body,
        out_shape=[
            jax.ShapeDtypeStruct((T, R), jnp.int32),
            jax.ShapeDtypeStruct((2 * DV, F), jnp.float32),
            jax.ShapeDtypeStruct((T, F), jnp.float32),
        ],
    )(x2, vel_table, ctrl_table)


def _sc_body(cidx_hbm, table2_hbm, rows_hbm, cidx_v, buf0, buf1,
             gsem0, gsem1):
    cid = lax.axis_index("c")
    sid = lax.axis_index("s")
    wid = cid * NSUB + sid
    t0 = wid * T_PER_W
    buf = (buf0, buf1)
    gsem = (gsem0, gsem1)

    pltpu.sync_copy(cidx_hbm.at[pl.ds(t0 * R, T_PER_W * R)], cidx_v)

    def gather(u, k):
        idx = cidx_v.at[pl.ds(u * R, R)]
        return pltpu.make_async_copy(table2_hbm.at[idx], buf[k], gsem[k])

    gather(0, 0).start()
    gather(1, 1).start()

    def pair(p, carry):
        for k in range(2):
            u = p * 2 + k
            gather(u, k).wait()
            pltpu.sync_copy(buf[k], rows_hbm.at[t0 + u])

            @pl.when(u + 2 < T_PER_W)
            def _():
                gather(u + 2, k).start()
        return carry

    lax.fori_loop(0, T_PER_W // 2, pair, 0)


@functools.cache
def _sc_gather():
    return pl.kernel(
        _sc_body,
        out_type=jax.ShapeDtypeStruct((T, R, F), jnp.float32),
        mesh=plsc.VectorSubcoreMesh(core_axis_name="c", subcore_axis_name="s",
                                    num_cores=NCORE, num_subcores=NSUB),
        scratch_types=[
            pltpu.VMEM((T_PER_W * R,), jnp.int32),
            pltpu.VMEM((R, F), jnp.float32),
            pltpu.VMEM((R, F), jnp.float32),
            pltpu.SemaphoreType.DMA,
            pltpu.SemaphoreType.DMA,
        ],
    )


def _add_body(rows_ref, pe_ref, o_ref):
    r = rows_ref[...]
    p = pe_ref[...]
    y = r + p[:, None, :]
    o_ref[...] = y.reshape(TB, B, NSLOT, F)


def _pe_add(rows3, pe2):
    return pl.pallas_call(
        _add_body,
        grid=(T // TB,),
        in_specs=[
            pl.BlockSpec((TB, R, F), lambda i: (i, 0, 0)),
            pl.BlockSpec((TB, F), lambda i: (i, 0)),
        ],
        out_specs=pl.BlockSpec((TB, B, NSLOT, F), lambda i: (i, 0, 0, 0)),
        out_shape=jax.ShapeDtypeStruct((T, B, NSLOT, F), jnp.float32),
    )(rows3, pe2)


def kernel(x, vel_table, ctrl_table):
    x2 = x.reshape(T, R).astype(jnp.int32)
    cidx, table2, pe2 = _prep(x2, vel_table, ctrl_table)
    rows3 = _sc_gather()(cidx.reshape(T * R), table2)
    return _pe_add(rows3, pe2)
